# baseline (speedup 1.0000x reference)
import functools

import jax
import jax.numpy as jnp
from jax import lax
from jax.experimental import pallas as pl
from jax.experimental.pallas import tpu as pltpu
from jax.experimental.pallas import tpu_sc as plsc

_LN09 = -0.10536051565782628
_HI = lax.Precision.HIGHEST
_NT = (((1,), (1,)), ((), ()))

_NSUB = 16
_RPS = 512 // _NSUB


def _tc1_kernel(lufeat_ref, w1_ref, b1_ref, w2_ref, b2_ref,
                lbl_ref, start_ref, mbank_ref,
                h_ref, bank_ref, rows_ref):
    f32 = jnp.float32
    h1 = lax.dot_general(lufeat_ref[...], w1_ref[...], _NT,
                         preferred_element_type=f32) + b1_ref[...][None, :]
    h2 = lax.dot_general(h1, w2_ref[...], _NT,
                         preferred_element_type=f32) + b2_ref[...][None, :]
    nrm = jnp.sqrt(jnp.sum(h2 * h2, axis=1, keepdims=True))
    h = h2 / jnp.maximum(nrm, 1e-12)
    h_ref[...] = h

    @pl.when(pl.program_id(0) == 0)
    def _():
        lfeat = h
        lblr = lbl_ref[...][None, :]
        lblc = jnp.transpose(lblr)
        startr = start_ref[...][None, :]
        startc = jnp.transpose(startr)
        match = (lblc == lblr)
        ii = lax.broadcasted_iota(jnp.int32, (512, 512), 0)
        jj = lax.broadcasted_iota(jnp.int32, (512, 512), 1)
        pc = jnp.sum(jnp.where(match & (jj <= ii), 1.0, 0.0), axis=1, keepdims=True)
        cnt = jnp.sum(jnp.where(match, 1.0, 0.0), axis=1, keepdims=True)
        r = cnt - pc
        onehot_i = (lblc == jj)
        st_i = jnp.sum(jnp.where(onehot_i, startr, 0.0), axis=1,
                       keepdims=True)
        first = (pc == 1.0) & (st_i == 0.0)
        coeff = jnp.exp(r * _LN09) * jnp.where(first, 1.0, 0.1)
        rows = coeff * lfeat
        rows_ref[0] = rows[:, :128]
        rows_ref[1] = rows[:, 128:]
        onehot_t = (ii == lblr)
        cnt_c = jnp.sum(jnp.where(onehot_t, 1.0, 0.0), axis=1, keepdims=True)
        base = jnp.where((startc == 0.0) & (cnt_c > 0.0), 0.0,
                         jnp.exp(cnt_c * _LN09))
        bank = base * mbank_ref[...]
        bank_ref[0] = bank[:, :128]
        bank_ref[1] = bank[:, 128:]


def _sc_scatter(bank_hbm, rows_hbm, idx_hbm, out_hbm, idx_v, rows_v, bank_sh, sem):
    c = lax.axis_index("c")
    s = lax.axis_index("s")
    rb = s * _RPS
    cp1 = pltpu.async_copy(bank_hbm.at[c, pl.ds(rb, _RPS)],
                           bank_sh.at[pl.ds(rb, _RPS)], sem)
    cp2 = pltpu.async_copy(idx_hbm.at[pl.ds(rb, _RPS)], idx_v, sem)
    cp3 = pltpu.async_copy(rows_hbm.at[c, pl.ds(rb, _RPS)], rows_v, sem)
    cp1.wait()
    cp2.wait()
    cp3.wait()
    plsc.subcore_barrier()
    pltpu.sync_copy(rows_v, bank_sh.at[idx_v], add=True)
    plsc.subcore_barrier()
    pltpu.sync_copy(bank_sh.at[pl.ds(rb, _RPS)], out_hbm.at[c, pl.ds(rb, _RPS)])


def _tc2_kernel(h_ref, mbucs_ref, w3_ref, b3_ref, g1_ref, be1_ref,
                w4_ref, b4_ref, g2_ref, be2_ref, w5_ref, b5_ref,
                lsc_ref, usc_ref, mbu_ref):
    f32 = jnp.float32
    h = h_ref[...]
    ufeat = h[512:]
    mbu = jnp.concatenate([mbucs_ref[0], mbucs_ref[1]], axis=1)
    mbu_ref[...] = mbu

    g = lax.dot_general(h, mbu, _NT, preferred_element_type=f32,
                        precision=_HI)
    mn2 = lax.dot_general(jnp.ones((1, 256), f32), mbu * mbu, _NT,
                          preferred_element_type=f32, precision=_HI)
    d2 = jnp.maximum((1.0 + mn2) - 2.0 * g, 0.0)
    lsc_ref[...] = jnp.transpose(jnp.sqrt(jnp.min(d2[:512], axis=1, keepdims=True)))
    um = jnp.sqrt(d2[512:])

    w3 = w3_ref[...]
    z = (lax.dot_general(ufeat, w3[:, :256], _NT, preferred_element_type=f32)
         + lax.dot_general(um, w3[:, 256:], _NT, preferred_element_type=f32)
         + b3_ref[...][None, :])
    m1 = jnp.mean(z, axis=0, keepdims=True)
    v1 = jnp.mean((z - m1) * (z - m1), axis=0, keepdims=True)
    u1 = jnp.maximum(g1_ref[...][None, :] * (z - m1) / jnp.sqrt(v1 + 1e-5)
                     + be1_ref[...][None, :], 0.0)
    z2 = (lax.dot_general(u1, w4_ref[...], _NT, preferred_element_type=f32)
          + b4_ref[...][None, :])
    m2 = jnp.mean(z2, axis=0, keepdims=True)
    v2 = jnp.mean((z2 - m2) * (z2 - m2), axis=0, keepdims=True)
    u2 = jnp.maximum(g2_ref[...][None, :] * (z2 - m2) / jnp.sqrt(v2 + 1e-5)
                     + be2_ref[...][None, :], 0.0)
    ust = lax.dot_general(w5_ref[...], u2, _NT, preferred_element_type=f32)
    usc_ref[...] = jnp.transpose(ust) + b5_ref[...][None, :]


@functools.partial(jax.jit, static_argnames=("interpret",))
def kernel(lufeat, llabel, mbank, start, W1, b1, W2, b2, W3, b3, W4, b4,
           W5, b5, g1, be1, g2, be2, interpret=False):
    f32 = jnp.float32
    lbl = llabel.astype(jnp.int32)
    _const = lambda *zeros: (lambda i: zeros)
    h, bank_cs, rows_cs = pl.pallas_call(
        _tc1_kernel,
        grid=(2,),
        in_specs=[
            pl.BlockSpec((512, 1024), lambda i: (i, 0)),
            pl.BlockSpec((512, 1024), _const(0, 0)),
            pl.BlockSpec((512,), _const(0)),
            pl.BlockSpec((256, 512), _const(0, 0)),
            pl.BlockSpec((256,), _const(0)),
            pl.BlockSpec((512,), _const(0)),
            pl.BlockSpec((512,), _const(0)),
            pl.BlockSpec((512, 256), _const(0, 0)),
        ],
        out_specs=[
            pl.BlockSpec((512, 256), lambda i: (i, 0)),
            pl.BlockSpec((2, 512, 128), _const(0, 0, 0)),
            pl.BlockSpec((2, 512, 128), _const(0, 0, 0)),
        ],
        out_shape=(
            jax.ShapeDtypeStruct((1024, 256), f32),
            jax.ShapeDtypeStruct((2, 512, 128), f32),
            jax.ShapeDtypeStruct((2, 512, 128), f32),
        ),
        interpret=interpret,
    )(lufeat, W1, b1, W2, b2, lbl, start, mbank)

    mesh = plsc.VectorSubcoreMesh(core_axis_name="c", subcore_axis_name="s")
    out_cs = pl.kernel(
        _sc_scatter,
        mesh=mesh,
        out_type=jax.ShapeDtypeStruct((2, 512, 128), f32),
        scratch_types=[
            pltpu.VMEM((_RPS,), jnp.int32),
            pltpu.VMEM((_RPS, 128), f32),
            pltpu.VMEM_SHARED((512, 128), f32),
            pltpu.SemaphoreType.DMA,
        ],
    )(bank_cs, rows_cs, lbl)

    lsc, usc, mbu = pl.pallas_call(
        _tc2_kernel,
        out_shape=(
            jax.ShapeDtypeStruct((1, 512), f32),
            jax.ShapeDtypeStruct((512, 1), f32),
            jax.ShapeDtypeStruct((512, 256), f32),
        ),
        interpret=interpret,
    )(h, out_cs, W3, b3, g1, be1, W4, b4, g2, be2,
      W5, b5)
    return (lsc.reshape(512), usc, mbu)


# speedup vs baseline: 1.0158x; 1.0158x over previous
import functools

import jax
import jax.numpy as jnp
from jax import lax
from jax.experimental import pallas as pl
from jax.experimental.pallas import tpu as pltpu
from jax.experimental.pallas import tpu_sc as plsc

_LN09 = -0.10536051565782628
_HI = lax.Precision.HIGHEST
_NT = (((1,), (1,)), ((), ()))

_NSUB = 16
_RPS = 512 // _NSUB


def _tc1_kernel(lufeat_ref, w1_ref, b1_ref, w2_ref, b2_ref,
                lbl_ref, start_ref, mbank_ref,
                h_ref, bank_ref, rows_ref):
    f32 = jnp.float32
    h1 = lax.dot_general(lufeat_ref[...], w1_ref[...], _NT,
                         preferred_element_type=f32) + b1_ref[...][None, :]
    h2 = lax.dot_general(h1, w2_ref[...], _NT,
                         preferred_element_type=f32) + b2_ref[...][None, :]
    nrm = jnp.sqrt(jnp.sum(h2 * h2, axis=1, keepdims=True))
    h = h2 / jnp.maximum(nrm, 1e-12)
    h_ref[...] = h

    @pl.when(pl.program_id(0) == 0)
    def _():
        lfeat = h
        lblr = lbl_ref[...][None, :]
        lblc = jnp.transpose(lblr)
        startr = start_ref[...][None, :]
        startc = jnp.transpose(startr)
        match = (lblc == lblr)
        ii = lax.broadcasted_iota(jnp.int32, (512, 512), 0)
        jj = lax.broadcasted_iota(jnp.int32, (512, 512), 1)
        pc = jnp.sum(jnp.where(match & (jj <= ii), 1.0, 0.0), axis=1, keepdims=True)
        cnt = jnp.sum(jnp.where(match, 1.0, 0.0), axis=1, keepdims=True)
        r = cnt - pc
        onehot_i = (lblc == jj)
        st_i = jnp.sum(jnp.where(onehot_i, startr, 0.0), axis=1,
                       keepdims=True)
        first = (pc == 1.0) & (st_i == 0.0)
        coeff = jnp.exp(r * _LN09) * jnp.where(first, 1.0, 0.1)
        rows = coeff * lfeat
        rows_ref[0] = rows[:, :128]
        rows_ref[1] = rows[:, 128:]
        onehot_t = (ii == lblr)
        cnt_c = jnp.sum(jnp.where(onehot_t, 1.0, 0.0), axis=1, keepdims=True)
        base = jnp.where((startc == 0.0) & (cnt_c > 0.0), 0.0,
                         jnp.exp(cnt_c * _LN09))
        bank = base * mbank_ref[...]
        bank_ref[0] = bank[:, :128]
        bank_ref[1] = bank[:, 128:]


def _sc_scatter(bank_hbm, rows_hbm, idx_hbm, out_hbm, idx_v, rows_v, bank_sh, sem):
    c = lax.axis_index("c")
    s = lax.axis_index("s")
    rb = s * _RPS
    cp1 = pltpu.async_copy(bank_hbm.at[c, pl.ds(rb, _RPS)],
                           bank_sh.at[pl.ds(rb, _RPS)], sem)
    cp2 = pltpu.async_copy(idx_hbm.at[pl.ds(rb, _RPS)], idx_v, sem)
    cp3 = pltpu.async_copy(rows_hbm.at[c, pl.ds(rb, _RPS)], rows_v, sem)
    cp1.wait()
    cp2.wait()
    cp3.wait()
    plsc.subcore_barrier()
    pltpu.sync_copy(rows_v, bank_sh.at[idx_v], add=True)
    plsc.subcore_barrier()
    pltpu.sync_copy(bank_sh.at[pl.ds(rb, _RPS)], out_hbm.at[c, pl.ds(rb, _RPS)])


def _tc2_kernel(h_ref, mbucs_ref, w3_ref, b3_ref, g1_ref, be1_ref,
                w4_ref, b4_ref, g2_ref, be2_ref, w5_ref, b5_ref,
                lsc_ref, usc_ref, mbu_ref):
    f32 = jnp.float32
    h = h_ref[...]
    ufeat = h[512:]
    mbu = jnp.concatenate([mbucs_ref[0], mbucs_ref[1]], axis=1)
    mbu_ref[...] = mbu

    g = lax.dot_general(h, mbu, _NT, preferred_element_type=f32,
                        precision=_HI)
    mn2 = lax.dot_general(jnp.ones((1, 256), f32), mbu * mbu, _NT,
                          preferred_element_type=f32, precision=_HI)
    d2 = jnp.maximum((1.0 + mn2) - 2.0 * g, 0.0)
    lsc_ref[...] = jnp.transpose(jnp.sqrt(jnp.min(d2[:512], axis=1, keepdims=True)))
    um = jnp.sqrt(d2[512:])

    w3 = w3_ref[...]
    z = (lax.dot_general(ufeat, w3[:, :256], _NT, preferred_element_type=f32)
         + lax.dot_general(um, w3[:, 256:], _NT, preferred_element_type=f32)
         + b3_ref[...][None, :])
    m1 = jnp.mean(z, axis=0, keepdims=True)
    v1 = jnp.mean((z - m1) * (z - m1), axis=0, keepdims=True)
    u1 = jnp.maximum(g1_ref[...][None, :] * (z - m1) / jnp.sqrt(v1 + 1e-5)
                     + be1_ref[...][None, :], 0.0)
    z2 = (lax.dot_general(u1, w4_ref[...], _NT, preferred_element_type=f32)
          + b4_ref[...][None, :])
    m2 = jnp.mean(z2, axis=0, keepdims=True)
    v2 = jnp.mean((z2 - m2) * (z2 - m2), axis=0, keepdims=True)
    u2 = jnp.maximum(g2_ref[...][None, :] * (z2 - m2) / jnp.sqrt(v2 + 1e-5)
                     + be2_ref[...][None, :], 0.0)
    usc_ref[...] = jnp.dot(u2, w5_ref[...], preferred_element_type=f32) + b5_ref[...][None, :]


@functools.partial(jax.jit, static_argnames=("interpret",))
def kernel(lufeat, llabel, mbank, start, W1, b1, W2, b2, W3, b3, W4, b4,
           W5, b5, g1, be1, g2, be2, interpret=False):
    f32 = jnp.float32
    lbl = llabel.astype(jnp.int32)
    _const = lambda *zeros: (lambda i: zeros)
    h, bank_cs, rows_cs = pl.pallas_call(
        _tc1_kernel,
        grid=(2,),
        in_specs=[
            pl.BlockSpec((512, 1024), lambda i: (i, 0)),
            pl.BlockSpec((512, 1024), _const(0, 0)),
            pl.BlockSpec((512,), _const(0)),
            pl.BlockSpec((256, 512), _const(0, 0)),
            pl.BlockSpec((256,), _const(0)),
            pl.BlockSpec((512,), _const(0)),
            pl.BlockSpec((512,), _const(0)),
            pl.BlockSpec((512, 256), _const(0, 0)),
        ],
        out_specs=[
            pl.BlockSpec((512, 256), lambda i: (i, 0)),
            pl.BlockSpec((2, 512, 128), _const(0, 0, 0)),
            pl.BlockSpec((2, 512, 128), _const(0, 0, 0)),
        ],
        out_shape=(
            jax.ShapeDtypeStruct((1024, 256), f32),
            jax.ShapeDtypeStruct((2, 512, 128), f32),
            jax.ShapeDtypeStruct((2, 512, 128), f32),
        ),
        interpret=interpret,
    )(lufeat, W1, b1, W2, b2, lbl, start, mbank)

    mesh = plsc.VectorSubcoreMesh(core_axis_name="c", subcore_axis_name="s")
    out_cs = pl.kernel(
        _sc_scatter,
        mesh=mesh,
        out_type=jax.ShapeDtypeStruct((2, 512, 128), f32),
        scratch_types=[
            pltpu.VMEM((_RPS,), jnp.int32),
            pltpu.VMEM((_RPS, 128), f32),
            pltpu.VMEM_SHARED((512, 128), f32),
            pltpu.SemaphoreType.DMA,
        ],
    )(bank_cs, rows_cs, lbl)

    lsc, usc, mbu = pl.pallas_call(
        _tc2_kernel,
        out_shape=(
            jax.ShapeDtypeStruct((1, 512), f32),
            jax.ShapeDtypeStruct((512, 1), f32),
            jax.ShapeDtypeStruct((512, 256), f32),
        ),
        interpret=interpret,
    )(h, out_cs, W3, b3, g1, be1, W4, b4, g2, be2,
      W5.reshape(64, 1), b5)
    return (lsc.reshape(512), usc, mbu)
